# baseline (speedup 1.0000x reference)
import jax
import jax.numpy as jnp
from jax.experimental import pallas as pl
from jax.experimental.pallas import tpu as pltpu

N = 100000
D = 128
TILE = 2000


def _body(x_ref, w1_ref, b1_ref, w2_ref, b2_ref,
          wt_ref, bt_ref, ws_ref, bs_ref,
          wtf_ref, btf_ref, wa_ref, ba_ref,
          ot_ref, os_ref, otf_ref, oa_ref):
    x = x_ref[...]
    h = jnp.maximum(
        jnp.dot(x, w1_ref[...], preferred_element_type=jnp.float32) + b1_ref[...],
        0.0)
    g = jnp.dot(h, w2_ref[...], preferred_element_type=jnp.float32) + b2_ref[...]
    ot_ref[...] = jnp.dot(g, wt_ref[...], preferred_element_type=jnp.float32) + bt_ref[...]
    os_ref[...] = jnp.dot(g, ws_ref[...], preferred_element_type=jnp.float32) + bs_ref[...]
    otf_ref[...] = jnp.dot(g, wtf_ref[...], preferred_element_type=jnp.float32) + btf_ref[...]
    oa_ref[...] = jnp.dot(g, wa_ref[...], preferred_element_type=jnp.float32) + ba_ref[...]


@jax.jit
def kernel(x, W1, b1, W2, b2, Wt, bt, Ws, bs, Wtf, btf, Wa, ba):
    n = x.shape[0]
    grid = (n // TILE,)

    def full(a):
        return pl.BlockSpec(a.shape, lambda i: (0,) * a.ndim)

    b1r = b1.reshape(1, -1)
    b2r = b2.reshape(1, -1)
    btr = bt.reshape(1, -1)
    bsr = bs.reshape(1, -1)
    btfr = btf.reshape(1, -1)
    bar = ba.reshape(1, -1)

    row = lambda d: pl.BlockSpec((TILE, d), lambda i: (i, 0))

    out_shapes = (
        jax.ShapeDtypeStruct((n, Wt.shape[1]), jnp.float32),
        jax.ShapeDtypeStruct((n, Ws.shape[1]), jnp.float32),
        jax.ShapeDtypeStruct((n, Wtf.shape[1]), jnp.float32),
        jax.ShapeDtypeStruct((n, Wa.shape[1]), jnp.float32),
    )

    return pl.pallas_call(
        _body,
        grid=grid,
        in_specs=[
            row(D),
            full(W1), full(b1r), full(W2), full(b2r),
            full(Wt), full(btr), full(Ws), full(bsr),
            full(Wtf), full(btfr), full(Wa), full(bar),
        ],
        out_specs=(
            row(Wt.shape[1]), row(Ws.shape[1]),
            row(Wtf.shape[1]), row(Wa.shape[1]),
        ),
        out_shape=out_shapes,
        compiler_params=pltpu.CompilerParams(
            dimension_semantics=("parallel",),
        ),
    )(x, W1, b1r, W2, b2r, Wt, btr, Ws, bsr, Wtf, btfr, Wa, bar)


# speedup vs baseline: 1.0108x; 1.0108x over previous
import jax
import jax.numpy as jnp
from jax.experimental import pallas as pl
from jax.experimental.pallas import tpu as pltpu

N = 100000
D = 128
TILE = 4000


def _body(x_ref, w1_ref, b1_ref, w2_ref, b2_ref,
          wt_ref, bt_ref, ws_ref, bs_ref,
          wtf_ref, btf_ref, wa_ref, ba_ref,
          ot_ref, os_ref, otf_ref, oa_ref):
    x = x_ref[...]
    h = jnp.maximum(
        jnp.dot(x, w1_ref[...], preferred_element_type=jnp.float32) + b1_ref[...],
        0.0)
    g = jnp.dot(h, w2_ref[...], preferred_element_type=jnp.float32) + b2_ref[...]
    ot_ref[...] = jnp.dot(g, wt_ref[...], preferred_element_type=jnp.float32) + bt_ref[...]
    os_ref[...] = jnp.dot(g, ws_ref[...], preferred_element_type=jnp.float32) + bs_ref[...]
    otf_ref[...] = jnp.dot(g, wtf_ref[...], preferred_element_type=jnp.float32) + btf_ref[...]
    oa_ref[...] = jnp.dot(g, wa_ref[...], preferred_element_type=jnp.float32) + ba_ref[...]


@jax.jit
def kernel(x, W1, b1, W2, b2, Wt, bt, Ws, bs, Wtf, btf, Wa, ba):
    n = x.shape[0]
    grid = (n // TILE,)

    def full(a):
        return pl.BlockSpec(a.shape, lambda i: (0,) * a.ndim)

    b1r = b1.reshape(1, -1)
    b2r = b2.reshape(1, -1)
    btr = bt.reshape(1, -1)
    bsr = bs.reshape(1, -1)
    btfr = btf.reshape(1, -1)
    bar = ba.reshape(1, -1)

    row = lambda d: pl.BlockSpec((TILE, d), lambda i: (i, 0))

    out_shapes = (
        jax.ShapeDtypeStruct((n, Wt.shape[1]), jnp.float32),
        jax.ShapeDtypeStruct((n, Ws.shape[1]), jnp.float32),
        jax.ShapeDtypeStruct((n, Wtf.shape[1]), jnp.float32),
        jax.ShapeDtypeStruct((n, Wa.shape[1]), jnp.float32),
    )

    return pl.pallas_call(
        _body,
        grid=grid,
        in_specs=[
            row(D),
            full(W1), full(b1r), full(W2), full(b2r),
            full(Wt), full(btr), full(Ws), full(bsr),
            full(Wtf), full(btfr), full(Wa), full(bar),
        ],
        out_specs=(
            row(Wt.shape[1]), row(Ws.shape[1]),
            row(Wtf.shape[1]), row(Wa.shape[1]),
        ),
        out_shape=out_shapes,
        compiler_params=pltpu.CompilerParams(
            dimension_semantics=("parallel",),
        ),
    )(x, W1, b1r, W2, b2r, Wt, btr, Ws, bsr, Wtf, btfr, Wa, bar)


# speedup vs baseline: 3.3135x; 3.2780x over previous
import jax
import jax.numpy as jnp
from jax import lax
from jax.experimental import pallas as pl
from jax.experimental.pallas import tpu as pltpu

D = 128
TILE = 2048

_DN_B1 = (((0,), (1,)), ((), ()))
_DN_B0 = (((0,), (0,)), ((), ()))


def _body(x_ref, w1_ref, b1_ref, w2_ref, b2_ref,
          wt_ref, bt_ref, ws_ref, bs_ref,
          wtf_ref, btf_ref, wa_ref, ba_ref,
          ot_ref, os_ref, otf_ref, oa_ref):
    x = x_ref[...]
    hT = jnp.maximum(
        lax.dot_general(w1_ref[...], x, _DN_B1,
                        preferred_element_type=jnp.float32) + b1_ref[...],
        0.0)
    gT = lax.dot_general(w2_ref[...], hT, _DN_B0,
                         preferred_element_type=jnp.float32) + b2_ref[...]
    ot_ref[...] = lax.dot_general(wt_ref[...], gT, _DN_B0,
                                  preferred_element_type=jnp.float32) + bt_ref[...]
    os_ref[...] = lax.dot_general(ws_ref[...], gT, _DN_B0,
                                  preferred_element_type=jnp.float32) + bs_ref[...]
    otf_ref[...] = lax.dot_general(wtf_ref[...], gT, _DN_B0,
                                   preferred_element_type=jnp.float32) + btf_ref[...]
    oa_ref[...] = lax.dot_general(wa_ref[...], gT, _DN_B0,
                                  preferred_element_type=jnp.float32) + ba_ref[...]


@jax.jit
def kernel(x, W1, b1, W2, b2, Wt, bt, Ws, bs, Wtf, btf, Wa, ba):
    n = x.shape[0]
    grid = (pl.cdiv(n, TILE),)

    def full(a):
        return pl.BlockSpec(a.shape, lambda i: (0,) * a.ndim)

    b1c = b1.reshape(-1, 1)
    b2c = b2.reshape(-1, 1)
    btc = bt.reshape(-1, 1)
    bsc = bs.reshape(-1, 1)
    btfc = btf.reshape(-1, 1)
    bac = ba.reshape(-1, 1)

    col = lambda c: pl.BlockSpec((c, TILE), lambda i: (0, i))

    out_shapes = (
        jax.ShapeDtypeStruct((Wt.shape[1], n), jnp.float32),
        jax.ShapeDtypeStruct((Ws.shape[1], n), jnp.float32),
        jax.ShapeDtypeStruct((Wtf.shape[1], n), jnp.float32),
        jax.ShapeDtypeStruct((Wa.shape[1], n), jnp.float32),
    )

    otT, osT, otfT, oaT = pl.pallas_call(
        _body,
        grid=grid,
        in_specs=[
            pl.BlockSpec((TILE, D), lambda i: (i, 0)),
            full(W1), full(b1c), full(W2), full(b2c),
            full(Wt), full(btc), full(Ws), full(bsc),
            full(Wtf), full(btfc), full(Wa), full(bac),
        ],
        out_specs=(
            col(Wt.shape[1]), col(Ws.shape[1]),
            col(Wtf.shape[1]), col(Wa.shape[1]),
        ),
        out_shape=out_shapes,
        compiler_params=pltpu.CompilerParams(
            dimension_semantics=("arbitrary",),
        ),
    )(x, W1, b1c, W2, b2c, Wt, btc, Ws, bsc, Wtf, btfc, Wa, bac)

    return (otT.T, osT.T, otfT.T, oaT.T)


# speedup vs baseline: 4.1016x; 1.2378x over previous
import jax
import jax.numpy as jnp
from jax import lax
from jax.experimental import pallas as pl
from jax.experimental.pallas import tpu as pltpu

D = 128
TILE = 4096

_DN_B1 = (((0,), (1,)), ((), ()))
_DN_B0 = (((0,), (0,)), ((), ()))
_DN_STD = (((1,), (0,)), ((), ()))


def _body(x_ref, w1_ref, b1_ref, w2_ref, b2_ref,
          wt_ref, bt_ref, ws_ref, bs_ref,
          wtf_ref, btf_ref, wa_ref, ba_ref,
          ot_ref, os_ref, otf_ref, oa_ref):
    x = x_ref[...]
    hT = jnp.maximum(
        lax.dot_general(w1_ref[...], x, _DN_B1,
                        preferred_element_type=jnp.float32) + b1_ref[...],
        0.0)
    gT = lax.dot_general(w2_ref[...], hT, _DN_B0,
                         preferred_element_type=jnp.float32) + b2_ref[...]
    ot_ref[...] = lax.dot_general(wt_ref[...], gT, _DN_STD,
                                  preferred_element_type=jnp.float32) + bt_ref[...]
    os_ref[...] = lax.dot_general(ws_ref[...], gT, _DN_STD,
                                  preferred_element_type=jnp.float32) + bs_ref[...]
    otf_ref[...] = lax.dot_general(wtf_ref[...], gT, _DN_STD,
                                   preferred_element_type=jnp.float32) + btf_ref[...]
    oa_ref[...] = lax.dot_general(wa_ref[...], gT, _DN_STD,
                                  preferred_element_type=jnp.float32) + ba_ref[...]


@jax.jit
def kernel(x, W1, b1, W2, b2, Wt, bt, Ws, bs, Wtf, btf, Wa, ba):
    n = x.shape[0]
    grid = (pl.cdiv(n, TILE),)

    def full(a):
        return pl.BlockSpec(a.shape, lambda i: (0,) * a.ndim)

    b1c = b1.reshape(-1, 1)
    b2c = b2.reshape(-1, 1)
    btc = bt.reshape(-1, 1)
    bsc = bs.reshape(-1, 1)
    btfc = btf.reshape(-1, 1)
    bac = ba.reshape(-1, 1)

    col = lambda c: pl.BlockSpec((c, TILE), lambda i: (0, i))

    out_shapes = (
        jax.ShapeDtypeStruct((Wt.shape[1], n), jnp.float32),
        jax.ShapeDtypeStruct((Ws.shape[1], n), jnp.float32),
        jax.ShapeDtypeStruct((Wtf.shape[1], n), jnp.float32),
        jax.ShapeDtypeStruct((Wa.shape[1], n), jnp.float32),
    )

    WtT, WsT, WtfT, WaT = Wt.T, Ws.T, Wtf.T, Wa.T

    otT, osT, otfT, oaT = pl.pallas_call(
        _body,
        grid=grid,
        in_specs=[
            pl.BlockSpec((TILE, D), lambda i: (i, 0)),
            full(W1), full(b1c), full(W2), full(b2c),
            full(WtT), full(btc), full(WsT), full(bsc),
            full(WtfT), full(btfc), full(WaT), full(bac),
        ],
        out_specs=(
            col(Wt.shape[1]), col(Ws.shape[1]),
            col(Wtf.shape[1]), col(Wa.shape[1]),
        ),
        out_shape=out_shapes,
        compiler_params=pltpu.CompilerParams(
            dimension_semantics=("arbitrary",),
        ),
    )(x, W1, b1c, W2, b2c, WtT, btc, WsT, bsc, WtfT, btfc, WaT, bac)

    return (otT.T, osT.T, otfT.T, oaT.T)


# speedup vs baseline: 4.4540x; 1.0859x over previous
import jax
import jax.numpy as jnp
from jax import lax
from jax.experimental import pallas as pl
from jax.experimental.pallas import tpu as pltpu

D = 128
TILE = 4096

_OFF_A = 0
_OFF_T = 504
_OFF_S = 520
_OFF_TF = 544
_ROWS = 552

_DN_B1 = (((0,), (1,)), ((), ()))
_DN_STD = (((1,), (0,)), ((), ()))
_DN_RT = (((1,), (1,)), ((), ()))
_DN_COL = (((0,), (0,)), ((), ()))


def _prep_body(w2_ref, b1r_ref, b2r_ref,
               wat_ref, bar_ref, wtt_ref, btr_ref,
               wst_ref, bsr_ref, wtft_ref, btfr_ref,
               wall_ref, ball_ref, b1c_ref):
    w2 = w2_ref[...]
    b2r = b2r_ref[...]
    wall_ref[...] = jnp.zeros((_ROWS, D), jnp.float32)
    ball_ref[...] = jnp.zeros((_ROWS, 1), jnp.float32)

    def fold(wct_ref, bcr_ref, off, rows):
        wct = wct_ref[...]
        wall_ref[pl.ds(off, rows), :] = lax.dot_general(
            wct, w2, _DN_RT, preferred_element_type=jnp.float32)
        brow = lax.dot_general(b2r, wct, _DN_RT,
                               preferred_element_type=jnp.float32) + bcr_ref[...]
        ball_ref[pl.ds(off, rows), :] = brow.T

    fold(wat_ref, bar_ref, _OFF_A, 500)
    fold(wtt_ref, btr_ref, _OFF_T, 10)
    fold(wst_ref, bsr_ref, _OFF_S, 20)
    fold(wtft_ref, btfr_ref, _OFF_TF, 2)
    b1c_ref[...] = b1r_ref[...].T


def _body(x_ref, w1_ref, b1c_ref, wall_ref, ball_ref,
          ot_ref, os_ref, otf_ref, oa_ref):
    hT = jnp.maximum(
        lax.dot_general(w1_ref[...], x_ref[...], _DN_B1,
                        preferred_element_type=jnp.float32) + b1c_ref[...],
        0.0)
    res = lax.dot_general(wall_ref[...], hT, _DN_STD,
                          preferred_element_type=jnp.float32) + ball_ref[...]
    oa_ref[...] = res[_OFF_A:_OFF_A + 500, :]
    ot_ref[...] = res[_OFF_T:_OFF_T + 10, :]
    os_ref[...] = res[_OFF_S:_OFF_S + 20, :]
    otf_ref[...] = res[_OFF_TF:_OFF_TF + 2, :]


@jax.jit
def kernel(x, W1, b1, W2, b2, Wt, bt, Ws, bs, Wtf, btf, Wa, ba):
    n = x.shape[0]

    def full(a):
        return pl.BlockSpec(a.shape, lambda *_: (0,) * a.ndim)

    b1r = b1.reshape(1, -1)
    b2r = b2.reshape(1, -1)
    prep_in = (W2, b1r, b2r,
               Wa.T, ba.reshape(1, -1), Wt.T, bt.reshape(1, -1),
               Ws.T, bs.reshape(1, -1), Wtf.T, btf.reshape(1, -1))
    Wall, ball, b1c = pl.pallas_call(
        _prep_body,
        grid=(1,),
        in_specs=[full(a) for a in prep_in],
        out_specs=(
            pl.BlockSpec((_ROWS, D), lambda i: (0, 0)),
            pl.BlockSpec((_ROWS, 1), lambda i: (0, 0)),
            pl.BlockSpec((D, 1), lambda i: (0, 0)),
        ),
        out_shape=(
            jax.ShapeDtypeStruct((_ROWS, D), jnp.float32),
            jax.ShapeDtypeStruct((_ROWS, 1), jnp.float32),
            jax.ShapeDtypeStruct((D, 1), jnp.float32),
        ),
    )(*prep_in)

    col = lambda c: pl.BlockSpec((c, TILE), lambda i: (0, i))

    otT, osT, otfT, oaT = pl.pallas_call(
        _body,
        grid=(pl.cdiv(n, TILE),),
        in_specs=[
            pl.BlockSpec((TILE, D), lambda i: (i, 0)),
            full(W1), full(b1c), full(Wall), full(ball),
        ],
        out_specs=(
            col(Wt.shape[1]), col(Ws.shape[1]),
            col(Wtf.shape[1]), col(Wa.shape[1]),
        ),
        out_shape=(
            jax.ShapeDtypeStruct((Wt.shape[1], n), jnp.float32),
            jax.ShapeDtypeStruct((Ws.shape[1], n), jnp.float32),
            jax.ShapeDtypeStruct((Wtf.shape[1], n), jnp.float32),
            jax.ShapeDtypeStruct((Wa.shape[1], n), jnp.float32),
        ),
        compiler_params=pltpu.CompilerParams(
            dimension_semantics=("arbitrary",),
        ),
    )(x, W1, b1c, Wall, ball)

    return (otT.T, osT.T, otfT.T, oaT.T)


# speedup vs baseline: 4.5448x; 1.0204x over previous
import jax
import jax.numpy as jnp
from jax import lax
from jax.experimental import pallas as pl
from jax.experimental.pallas import tpu as pltpu

D = 128
TILE = 8192

_OFF_A = 0
_OFF_T = 504
_OFF_S = 520
_OFF_TF = 544
_ROWS = 552

_DN_B1 = (((0,), (1,)), ((), ()))
_DN_STD = (((1,), (0,)), ((), ()))
_DN_RT = (((1,), (1,)), ((), ()))
_DN_COL = (((0,), (0,)), ((), ()))


def _prep_body(w2_ref, b1r_ref, b2r_ref,
               wat_ref, bar_ref, wtt_ref, btr_ref,
               wst_ref, bsr_ref, wtft_ref, btfr_ref,
               wall_ref, ball_ref, b1c_ref):
    w2 = w2_ref[...]
    b2r = b2r_ref[...]
    wall_ref[...] = jnp.zeros((_ROWS, D), jnp.float32)
    ball_ref[...] = jnp.zeros((_ROWS, 1), jnp.float32)

    def fold(wct_ref, bcr_ref, off, rows):
        wct = wct_ref[...]
        wall_ref[pl.ds(off, rows), :] = lax.dot_general(
            wct, w2, _DN_RT, preferred_element_type=jnp.float32)
        brow = lax.dot_general(b2r, wct, _DN_RT,
                               preferred_element_type=jnp.float32) + bcr_ref[...]
        ball_ref[pl.ds(off, rows), :] = brow.T

    fold(wat_ref, bar_ref, _OFF_A, 500)
    fold(wtt_ref, btr_ref, _OFF_T, 10)
    fold(wst_ref, bsr_ref, _OFF_S, 20)
    fold(wtft_ref, btfr_ref, _OFF_TF, 2)
    b1c_ref[...] = b1r_ref[...].T


def _body(x_ref, w1_ref, b1c_ref, wall_ref, ball_ref,
          ot_ref, os_ref, otf_ref, oa_ref):
    hT = jnp.maximum(
        lax.dot_general(w1_ref[...], x_ref[...], _DN_B1,
                        preferred_element_type=jnp.float32) + b1c_ref[...],
        0.0)
    res = lax.dot_general(wall_ref[...], hT, _DN_STD,
                          preferred_element_type=jnp.float32) + ball_ref[...]
    oa_ref[...] = res[_OFF_A:_OFF_A + 500, :]
    ot_ref[...] = res[_OFF_T:_OFF_T + 10, :]
    os_ref[...] = res[_OFF_S:_OFF_S + 20, :]
    otf_ref[...] = res[_OFF_TF:_OFF_TF + 2, :]


@jax.jit
def kernel(x, W1, b1, W2, b2, Wt, bt, Ws, bs, Wtf, btf, Wa, ba):
    n = x.shape[0]

    def full(a):
        return pl.BlockSpec(a.shape, lambda *_: (0,) * a.ndim)

    b1r = b1.reshape(1, -1)
    b2r = b2.reshape(1, -1)
    prep_in = (W2, b1r, b2r,
               Wa.T, ba.reshape(1, -1), Wt.T, bt.reshape(1, -1),
               Ws.T, bs.reshape(1, -1), Wtf.T, btf.reshape(1, -1))
    Wall, ball, b1c = pl.pallas_call(
        _prep_body,
        grid=(1,),
        in_specs=[full(a) for a in prep_in],
        out_specs=(
            pl.BlockSpec((_ROWS, D), lambda i: (0, 0)),
            pl.BlockSpec((_ROWS, 1), lambda i: (0, 0)),
            pl.BlockSpec((D, 1), lambda i: (0, 0)),
        ),
        out_shape=(
            jax.ShapeDtypeStruct((_ROWS, D), jnp.float32),
            jax.ShapeDtypeStruct((_ROWS, 1), jnp.float32),
            jax.ShapeDtypeStruct((D, 1), jnp.float32),
        ),
    )(*prep_in)

    col = lambda c: pl.BlockSpec((c, TILE), lambda i: (0, i))

    otT, osT, otfT, oaT = pl.pallas_call(
        _body,
        grid=(pl.cdiv(n, TILE),),
        in_specs=[
            pl.BlockSpec((TILE, D), lambda i: (i, 0)),
            full(W1), full(b1c), full(Wall), full(ball),
        ],
        out_specs=(
            col(Wt.shape[1]), col(Ws.shape[1]),
            col(Wtf.shape[1]), col(Wa.shape[1]),
        ),
        out_shape=(
            jax.ShapeDtypeStruct((Wt.shape[1], n), jnp.float32),
            jax.ShapeDtypeStruct((Ws.shape[1], n), jnp.float32),
            jax.ShapeDtypeStruct((Wtf.shape[1], n), jnp.float32),
            jax.ShapeDtypeStruct((Wa.shape[1], n), jnp.float32),
        ),
        compiler_params=pltpu.CompilerParams(
            dimension_semantics=("arbitrary",),
        ),
    )(x, W1, b1c, Wall, ball)

    return (otT.T, osT.T, otfT.T, oaT.T)
